# baseline (speedup 1.0000x reference)
import jax
import jax.numpy as jnp
from jax import lax
from jax.experimental import pallas as pl
from jax.experimental.pallas import tpu as pltpu

FINAL_ITER = 200000
THRESHOLD = 0.6

_ROWS = 128
_COLS = 100000
_BLOCK_ROWS = 8
_NBLK = _ROWS // _BLOCK_ROWS

_KS0 = 0
_KS1 = 12345
_KS2 = _KS0 ^ _KS1 ^ 0x1BD11BDA
_ROT0 = (13, 15, 26, 6)
_ROT1 = (17, 29, 16, 24)
_KS = (_KS0, _KS1, _KS2)


def _threefry_bits(n):
    x0 = jnp.uint32(_KS[0])
    x1 = n + jnp.uint32(_KS[1])
    rotations = (_ROT0, _ROT1)
    for i_round in range(5):
        for d in rotations[i_round % 2]:
            x0 = x0 + x1
            x1 = (x1 << jnp.uint32(d)) | (x1 >> jnp.uint32(32 - d))
            x1 = x0 ^ x1
        x0 = x0 + jnp.uint32(_KS[(i_round + 1) % 3])
        x1 = x1 + jnp.uint32((_KS[(i_round + 2) % 3] + i_round + 1) & 0xFFFFFFFF)
    return x0 ^ x1


def _body(t_ref, y_ref, thr_ref, o_ref):
    b = pl.program_id(0)
    thr = thr_ref[0]
    n = (
        lax.broadcasted_iota(jnp.uint32, (_BLOCK_ROWS, _COLS), 0)
        * jnp.uint32(_COLS)
        + lax.broadcasted_iota(jnp.uint32, (_BLOCK_ROWS, _COLS), 1)
        + (b * (_BLOCK_ROWS * _COLS)).astype(jnp.uint32)
    )
    mask = _threefry_bits(n) < thr
    o_ref[...] = jnp.where(mask, y_ref[...], t_ref[...])


def kernel(target, y, now_iter):
    k = 1.0
    c = (k - THRESHOLD) / FINAL_ITER
    sampling_prob = jnp.maximum(THRESHOLD, k - c * now_iter)
    p = 1.0 - sampling_prob
    thr = (jnp.ceil(p * 8388608.0).astype(jnp.uint32) << 9).reshape(1)

    return pl.pallas_call(
        _body,
        grid=(_NBLK,),
        in_specs=[
            pl.BlockSpec((_BLOCK_ROWS, _COLS), lambda b: (b, 0)),
            pl.BlockSpec((_BLOCK_ROWS, _COLS), lambda b: (b, 0)),
            pl.BlockSpec(memory_space=pltpu.SMEM),
        ],
        out_specs=pl.BlockSpec((_BLOCK_ROWS, _COLS), lambda b: (b, 0)),
        out_shape=jax.ShapeDtypeStruct((_ROWS, _COLS), jnp.float32),
        compiler_params=pltpu.CompilerParams(
            dimension_semantics=("parallel",),
        ),
    )(target, y, thr)
